# Initial kernel scaffold; baseline (speedup 1.0000x reference)
import jax
import jax.numpy as jnp
from jax.experimental import pallas as pl


def kernel(x, edge_index, batch, Wq, bq, Wk, bk, Wv, bv, Ws, bs):
    raise NotImplementedError("write your pallas kernel here")


# speedup vs baseline: 2.4488x; 2.4488x over previous
import functools

import jax
import jax.numpy as jnp
from jax import lax
from jax.experimental import pallas as pl
from jax.experimental.pallas import tpu as pltpu
from jax.experimental.pallas import tpu_sc as plsc

N_NODES = 10000
N_GRAPHS = 64
D = 128
E_TOTAL = 320000

NC = 2
NS = 16
NW = NC * NS
E_PER_W = E_TOTAL // NW
CHUNK = 80
N_CHUNKS = E_PER_W // CHUNK
GROUPS = CHUNK // 16
INV_SQRT_D = float(1.0 / (D ** 0.5))

BM = 1000


def _proj_body(x_ref, w_ref, b_ref, o_ref):
    o_ref[...] = (
        jnp.dot(x_ref[...], w_ref[...], preferred_element_type=jnp.float32)
        + b_ref[...][0][None, :]
    )


def _project(x, wcat, bcat8):
    return pl.pallas_call(
        _proj_body,
        grid=(N_NODES // BM,),
        in_specs=[
            pl.BlockSpec((BM, D), lambda i: (i, 0)),
            pl.BlockSpec((D, 4 * D), lambda i: (0, 0)),
            pl.BlockSpec((8, 4 * D), lambda i: (0, 0)),
        ],
        out_specs=pl.BlockSpec((BM, 4 * D), lambda i: (i, 0)),
        out_shape=jax.ShapeDtypeStruct((N_NODES, 4 * D), jnp.float32),
    )(x, wcat, bcat8)


def _sc_body(q_hbm, k_hbm, v_hbm, src_hbm, dst_hbm, agg_out, den_out,
             src_v, dst_v, qrows, krows, vrows, ex_v, zrow, zd,
             agg_sh, den_sh):
    cid = lax.axis_index("c")
    sid = lax.axis_index("s")
    wid = sid * NC + cid
    lanes = lax.iota(jnp.int32, 16)
    zero16 = jnp.zeros((16,), jnp.float32)

    @pl.when(sid == 0)
    def _():
        def zr(r, carry):
            for j in range(8):
                zrow[r, pl.ds(j * 16, 16)] = zero16
            return carry
        lax.fori_loop(0, CHUNK, zr, 0)
        for g in range(GROUPS):
            zd[pl.ds(g * 16, 16)] = zero16

        def zs(c, carry):
            pltpu.sync_copy(zrow, agg_sh.at[pl.ds(c * CHUNK, CHUNK)])
            pltpu.sync_copy(zd, den_sh.at[pl.ds(c * CHUNK, CHUNK)])
            return carry
        lax.fori_loop(0, N_NODES // CHUNK, zs, 0)

    plsc.subcore_barrier()

    base_w = wid * E_PER_W

    def chunk_body(c, carry):
        base = base_w + c * CHUNK
        pltpu.sync_copy(src_hbm.at[pl.ds(base, CHUNK)], src_v)
        pltpu.sync_copy(dst_hbm.at[pl.ds(base, CHUNK)], dst_v)
        pltpu.sync_copy(q_hbm.at[dst_v], qrows)
        pltpu.sync_copy(k_hbm.at[src_v], krows)
        pltpu.sync_copy(v_hbm.at[src_v], vrows)

        for g in range(GROUPS):
            le = g * 16 + lanes

            def dot_d(d, acc):
                dd = jnp.full((16,), d, jnp.int32)
                return acc + (plsc.load_gather(qrows, [le, dd])
                              * plsc.load_gather(krows, [le, dd]))
            s = lax.fori_loop(0, D, dot_d, zero16)
            exv = jnp.exp(s * INV_SQRT_D)
            ex_v[pl.ds(g * 16, 16)] = exv

            def vscale(d, carry2):
                dd = jnp.full((16,), d, jnp.int32)
                plsc.store_scatter(
                    vrows, [le, dd],
                    plsc.load_gather(vrows, [le, dd]) * exv)
                return carry2
            lax.fori_loop(0, D, vscale, 0)

        pltpu.sync_copy(ex_v, den_sh.at[dst_v], add=True)
        pltpu.sync_copy(vrows, agg_sh.at[dst_v], add=True)
        return carry

    lax.fori_loop(0, N_CHUNKS, chunk_body, 0)
    plsc.subcore_barrier()

    @pl.when(sid == 0)
    def _():
        pltpu.sync_copy(agg_sh, agg_out.at[cid])
        pltpu.sync_copy(den_sh, den_out.at[cid])


@functools.cache
def _sc_attention_kernel():
  return functools.partial(
    pl.kernel,
    mesh=plsc.VectorSubcoreMesh(
        core_axis_name="c", subcore_axis_name="s",
        num_cores=NC, num_subcores=NS),
    compiler_params=pltpu.CompilerParams(needs_layout_passes=False),
    out_type=[
        jax.ShapeDtypeStruct((NC, N_NODES, D), jnp.float32),
        jax.ShapeDtypeStruct((NC, N_NODES), jnp.float32),
    ],
    scratch_types=[
        pltpu.VMEM((CHUNK,), jnp.int32),
        pltpu.VMEM((CHUNK,), jnp.int32),
        pltpu.VMEM((CHUNK, D), jnp.float32),
        pltpu.VMEM((CHUNK, D), jnp.float32),
        pltpu.VMEM((CHUNK, D), jnp.float32),
        pltpu.VMEM((CHUNK,), jnp.float32),
        pltpu.VMEM((CHUNK, D), jnp.float32),
        pltpu.VMEM((CHUNK,), jnp.float32),
        pltpu.VMEM_SHARED((N_NODES, D), jnp.float32),
        pltpu.VMEM_SHARED((N_NODES,), jnp.float32),
    ],
  )(_sc_body)


def _pool_body(aggA_ref, aggB_ref, denA_ref, denB_ref, skip_ref, batch_ref,
               o_ref, acc, cnt):
    i = pl.program_id(0)

    @pl.when(i == 0)
    def _():
        acc[...] = jnp.zeros_like(acc)
        cnt[...] = jnp.zeros_like(cnt)

    den = denA_ref[0, 0] + denB_ref[0, 0]
    rows = (aggA_ref[...] + aggB_ref[...]) / (den[:, None] + 1e-16) + skip_ref[...]
    rows = jnp.maximum(rows, 0.0)
    b = batch_ref[0, 0]
    oh = (b[None, :] == lax.broadcasted_iota(jnp.int32, (N_GRAPHS, BM), 0)
          ).astype(jnp.float32)
    acc[...] += jnp.dot(oh, rows, preferred_element_type=jnp.float32)
    cnt[...] += jnp.sum(oh, axis=1)[:, None]

    @pl.when(i == (N_NODES // BM) - 1)
    def _():
        o_ref[...] = acc[...] / jnp.maximum(cnt[...], 1.0)


def _pool(aggA, aggB, den3A, den3B, skip, batch3):
    nb = N_NODES // BM
    return pl.pallas_call(
        _pool_body,
        grid=(nb,),
        in_specs=[
            pl.BlockSpec((BM, D), lambda i: (i, 0)),
            pl.BlockSpec((BM, D), lambda i: (i, 0)),
            pl.BlockSpec((1, 1, BM), lambda i: (i, 0, 0)),
            pl.BlockSpec((1, 1, BM), lambda i: (i, 0, 0)),
            pl.BlockSpec((BM, D), lambda i: (i, 0)),
            pl.BlockSpec((1, 1, BM), lambda i: (i, 0, 0)),
        ],
        out_specs=pl.BlockSpec((N_GRAPHS, D), lambda i: (0, 0)),
        out_shape=jax.ShapeDtypeStruct((N_GRAPHS, D), jnp.float32),
        scratch_shapes=[
            pltpu.VMEM((N_GRAPHS, D), jnp.float32),
            pltpu.VMEM((N_GRAPHS, D), jnp.float32),
        ],
    )(aggA, aggB, den3A, den3B, skip, batch3)


def kernel(x, edge_index, batch, Wq, bq, Wk, bk, Wv, bv, Ws, bs):
    wcat = jnp.concatenate([Wq, Wk, Wv, Ws], axis=1)
    bcat8 = jnp.tile(jnp.concatenate([bq, bk, bv, bs])[None, :], (8, 1))
    proj = _project(x, wcat, bcat8)
    q = proj[:, 0:D]
    k = proj[:, D:2 * D]
    v = proj[:, 2 * D:3 * D]
    skip = proj[:, 3 * D:4 * D]

    src = edge_index[0]
    dst = edge_index[1]
    agg2, den2 = _sc_attention_kernel()(q, k, v, src, dst)

    nb = N_NODES // BM
    den3A = den2[0].reshape(nb, 1, BM)
    den3B = den2[1].reshape(nb, 1, BM)
    batch3 = batch.reshape(nb, 1, BM)
    return _pool(agg2[0], agg2[1], den3A, den3B, skip, batch3)
